# baseline (speedup 1.0000x reference)
import functools

import jax
import jax.numpy as jnp
from jax import lax
from jax.experimental import pallas as pl
from jax.experimental.pallas import tpu as pltpu
from jax.experimental.pallas import tpu_sc as plsc

_K = 8192
_D = 32
_B = 8
_S = 1024
_M = _B * _S
_COMMIT = 0.25

_BM = 512
_BK = 1024
_BLK = 2048
_M_TILES = _M // _BM
_N_BLOCKS = _K // _BLK
_TILES_PER_BLK = _BLK // _BK

_NC = 2
_NS = 16
_NW = _NC * _NS
_ROWS_PER_W = _M // _NW
_CHUNK = 128
_NCHUNK = _ROWS_PER_W // _CHUNK


def _tile_argmin(xb, x2, w2_ref, wth_ref, wtl_ref, j):
    wth = wth_ref[:, j * _BK:(j + 1) * _BK]
    wtl = wtl_ref[:, j * _BK:(j + 1) * _BK]
    w2 = w2_ref[:, j * _BK:(j + 1) * _BK]
    m = lax.dot_general(xb, wth, (((1,), (0,)), ((), ())),
                        preferred_element_type=jnp.float32)
    m = m + lax.dot_general(xb, wtl, (((1,), (0,)), ((), ())),
                            preferred_element_type=jnp.float32)
    dist = (x2 + w2) - 2.0 * m
    tmin = jnp.min(dist, axis=1, keepdims=True)
    iota = lax.broadcasted_iota(jnp.int32, (_BM, _BK), 1)
    cand = jnp.where(dist == tmin, iota, _BK)
    tidx = jnp.min(cand, axis=1, keepdims=True) + j * _BK
    return tmin, tidx


def _argmin_body(xb_ref, x2_ref, w2_ref, wth_ref, wtl_ref, idx_ref, loss_ref):
    i = pl.program_id(0)
    xb = xb_ref[...]
    x2 = x2_ref[...]
    acc_v = None
    acc_i = None
    for blk in range(_N_BLOCKS):
        bv = None
        bi = None
        for t in range(_TILES_PER_BLK):
            tmin, tidx = _tile_argmin(xb, x2, w2_ref, wth_ref, wtl_ref,
                                      blk * _TILES_PER_BLK + t)
            if bv is None:
                bv, bi = tmin, tidx
            else:
                better = tmin < bv
                bv = jnp.where(better, tmin, bv)
                bi = jnp.where(better, tidx, bi)
        if acc_v is None:
            acc_v, acc_i = bv, bi
        else:
            acc_coarse = acc_v.astype(jnp.bfloat16).astype(jnp.float32)
            keep = acc_coarse <= bv
            acc_v = jnp.where(keep, acc_v, bv)
            acc_i = jnp.where(keep, acc_i, bi)
    idx_ref[...] = acc_i

    @pl.when(i == 0)
    def _init():
        loss_ref[...] = jnp.zeros_like(loss_ref)

    loss_ref[...] = loss_ref[...] + jnp.sum(acc_v)


_argmin_call = pl.pallas_call(
    _argmin_body,
    grid=(_M_TILES,),
    in_specs=[
        pl.BlockSpec((_BM, _D), lambda i: (i, 0)),
        pl.BlockSpec((_BM, 1), lambda i: (i, 0)),
        pl.BlockSpec((1, _K), lambda i: (0, 0)),
        pl.BlockSpec((_D, _K), lambda i: (0, 0)),
        pl.BlockSpec((_D, _K), lambda i: (0, 0)),
    ],
    out_specs=[
        pl.BlockSpec((_BM, 1), lambda i: (i, 0)),
        pl.BlockSpec((1, 1), lambda i: (0, 0)),
    ],
    out_shape=[
        jax.ShapeDtypeStruct((_M, 1), jnp.int32),
        jax.ShapeDtypeStruct((1, 1), jnp.float32),
    ],
    compiler_params=pltpu.CompilerParams(
        dimension_semantics=("arbitrary",),
    ),
)


@functools.partial(
    pl.kernel,
    mesh=plsc.VectorSubcoreMesh(core_axis_name="c", subcore_axis_name="s"),
    out_type=jax.ShapeDtypeStruct((_NW * _NCHUNK, _CHUNK, _D), jnp.float32),
    scratch_types=[
        pltpu.VMEM((_NCHUNK, _CHUNK), jnp.int32),
        pltpu.VMEM((_NCHUNK, _CHUNK, _D), jnp.float32),
        pltpu.VMEM((_NCHUNK, _CHUNK, _D), jnp.float32),
        pltpu.SemaphoreType.DMA,
    ],
    compiler_params=pltpu.CompilerParams(use_tc_tiling_on_sc=False),
)
def _sc_gather_st(w_hbm, idx_hbm, lat_hbm, out_hbm, idx_v, rows_v, lat_v, sem):
    wid = lax.axis_index("s") * _NC + lax.axis_index("c")
    base = wid * _NCHUNK
    pltpu.sync_copy(idx_hbm.at[pl.ds(base, _NCHUNK)], idx_v)
    copies = [
        pltpu.async_copy(w_hbm.at[idx_v.at[k]], rows_v.at[k], sem)
        for k in range(_NCHUNK)
    ]
    pltpu.sync_copy(lat_hbm.at[pl.ds(base, _NCHUNK)], lat_v)
    for cp in copies:
        cp.wait()

    def _st_row(i, _):
        for k in range(_NCHUNK):
            for c in range(_D // 16):
                sl = pl.ds(c * 16, 16)
                l = lat_v[k, i, sl]
                q = rows_v[k, i, sl]
                rows_v[k, i, sl] = l + (q - l)
        return 0

    lax.fori_loop(0, _CHUNK, _st_row, 0)
    pltpu.sync_copy(rows_v, out_hbm.at[pl.ds(base, _NCHUNK)])


def kernel(latents, W):
    flat = latents.reshape(-1, _D)
    x2 = jnp.sum(flat ** 2, axis=1, keepdims=True)
    w2 = jnp.sum(W ** 2, axis=1).reshape(1, _K)
    wt = W.T
    wt_hi = wt.astype(jnp.bfloat16)
    wt_lo = (wt - wt_hi.astype(jnp.float32)).astype(jnp.bfloat16)
    xb = flat.astype(jnp.bfloat16)
    idx2d, loss_sum = _argmin_call(xb, x2, w2, wt_hi, wt_lo)
    idx_chunks = idx2d.reshape(_NW * _NCHUNK, _CHUNK)
    lat_chunks = flat.reshape(_NW * _NCHUNK, _CHUNK, _D)
    qst = _sc_gather_st(W, idx_chunks, lat_chunks)
    mean_sq = loss_sum[0, 0] / (_M * _D)
    loss = _COMMIT * mean_sq + mean_sq
    indices = idx2d.reshape(_B, _S)
    return qst.reshape(_B, _S, _D), loss, indices
